# Initial kernel scaffold; baseline (speedup 1.0000x reference)
import jax
import jax.numpy as jnp
from jax.experimental import pallas as pl


def kernel(x, table):
    raise NotImplementedError("write your pallas kernel here")


# speedup vs baseline: 1.5613x; 1.5613x over previous
import functools

import jax
import jax.numpy as jnp
from jax import lax
from jax.experimental import pallas as pl
from jax.experimental.pallas import tpu as pltpu
from jax.experimental.pallas import tpu_sc as plsc

EMB_DIM = 32
NUM_WORKERS = 32
CHUNK = 1664


def _emb_body(x_hbm, table_hbm, out_hbm, idx_v, rows_v, sem):
    n_flat = out_hbm.shape[0]
    b_per_w = n_flat // NUM_WORKERS
    n_chunks = b_per_w // CHUNK
    wid = lax.axis_index("s") * 2 + lax.axis_index("c")
    base = wid * b_per_w

    @pl.loop(0, n_chunks)
    def _chunk(c):
        off = base + c * CHUNK
        pltpu.sync_copy(x_hbm.at[pl.ds(off, CHUNK)], idx_v)
        pltpu.async_copy(table_hbm.at[idx_v], rows_v, sem).wait()
        pltpu.sync_copy(rows_v, out_hbm.at[pl.ds(off, CHUNK)])


def kernel(x, table):
    batch, n_fields = x.shape
    n_flat = batch * n_fields
    x_flat = x.reshape(n_flat).astype(jnp.int32)

    mesh = plsc.VectorSubcoreMesh(core_axis_name="c", subcore_axis_name="s")
    emb = pl.kernel(
        _emb_body,
        out_type=jax.ShapeDtypeStruct((n_flat, EMB_DIM), jnp.float32),
        mesh=mesh,
        scratch_types=[
            pltpu.VMEM((CHUNK,), jnp.int32),
            pltpu.VMEM((CHUNK, EMB_DIM), jnp.float32),
            pltpu.SemaphoreType.DMA,
        ],
        compiler_params=pltpu.CompilerParams(use_tc_tiling_on_sc=False),
    )
    out_flat = emb(x_flat, table)
    return out_flat.reshape(batch, n_fields, EMB_DIM)


# speedup vs baseline: 1.5713x; 1.0064x over previous
import functools

import jax
import jax.numpy as jnp
from jax import lax
from jax.experimental import pallas as pl
from jax.experimental.pallas import tpu as pltpu
from jax.experimental.pallas import tpu_sc as plsc

EMB_DIM = 32
NUM_WORKERS = 32
CHUNK = 1664


def _emb_body(x_hbm, table_hbm, out_hbm, idx0, idx1, rows0, rows1, gsem, wsem):
    n_flat = out_hbm.shape[0]
    b_per_w = n_flat // NUM_WORKERS
    n_chunks = b_per_w // CHUNK
    wid = lax.axis_index("s") * 2 + lax.axis_index("c")
    base = wid * b_per_w

    idx = [idx0, idx1]
    rows = [rows0, rows1]
    gathers = [None] * n_chunks
    writes = [None] * n_chunks

    pltpu.sync_copy(x_hbm.at[pl.ds(base, CHUNK)], idx[0])
    gathers[0] = pltpu.async_copy(table_hbm.at[idx[0]], rows[0], gsem)
    for c in range(n_chunks):
        b = c & 1
        nb = 1 - b
        if c + 1 < n_chunks:
            if c >= 1:
                writes[c - 1].wait()
            pltpu.sync_copy(x_hbm.at[pl.ds(base + (c + 1) * CHUNK, CHUNK)], idx[nb])
            gathers[c + 1] = pltpu.async_copy(table_hbm.at[idx[nb]], rows[nb], gsem)
        gathers[c].wait()
        writes[c] = pltpu.async_copy(
            rows[b], out_hbm.at[pl.ds(base + c * CHUNK, CHUNK)], wsem)
    writes[n_chunks - 2].wait()
    writes[n_chunks - 1].wait()


def kernel(x, table):
    batch, n_fields = x.shape
    n_flat = batch * n_fields
    x_flat = x.reshape(n_flat).astype(jnp.int32)

    mesh = plsc.VectorSubcoreMesh(core_axis_name="c", subcore_axis_name="s")
    emb = pl.kernel(
        _emb_body,
        out_type=jax.ShapeDtypeStruct((n_flat, EMB_DIM), jnp.float32),
        mesh=mesh,
        scratch_types=[
            pltpu.VMEM((CHUNK,), jnp.int32),
            pltpu.VMEM((CHUNK,), jnp.int32),
            pltpu.VMEM((CHUNK, EMB_DIM), jnp.float32),
            pltpu.VMEM((CHUNK, EMB_DIM), jnp.float32),
            pltpu.SemaphoreType.DMA,
            pltpu.SemaphoreType.DMA,
        ],
        compiler_params=pltpu.CompilerParams(use_tc_tiling_on_sc=False),
    )
    out_flat = emb(x_flat, table)
    return out_flat.reshape(batch, n_fields, EMB_DIM)


# speedup vs baseline: 1.5746x; 1.0021x over previous
import functools

import jax
import jax.numpy as jnp
from jax import lax
from jax.experimental import pallas as pl
from jax.experimental.pallas import tpu as pltpu
from jax.experimental.pallas import tpu_sc as plsc

EMB_DIM = 32
NUM_WORKERS = 32
CHUNK = 1664
N_STREAMS = 8
SUB = CHUNK // N_STREAMS


def _emb_body(x_hbm, table_hbm, out_hbm, idx0, idx1, rows0, rows1, gsem, wsem):
    n_flat = out_hbm.shape[0]
    b_per_w = n_flat // NUM_WORKERS
    n_chunks = b_per_w // CHUNK
    wid = lax.axis_index("s") * 2 + lax.axis_index("c")
    base = wid * b_per_w

    idx = [idx0, idx1]
    rows = [rows0, rows1]
    gathers = [None] * n_chunks
    writes = [None] * n_chunks

    def start_gather(buf):
        return [
            pltpu.async_copy(
                table_hbm.at[idx[buf].at[pl.ds(g * SUB, SUB)]],
                rows[buf].at[pl.ds(g * SUB, SUB)],
                gsem,
            )
            for g in range(N_STREAMS)
        ]

    pltpu.sync_copy(x_hbm.at[pl.ds(base, CHUNK)], idx[0])
    gathers[0] = start_gather(0)
    for c in range(n_chunks):
        b = c & 1
        nb = 1 - b
        if c + 1 < n_chunks:
            if c >= 1:
                writes[c - 1].wait()
            pltpu.sync_copy(x_hbm.at[pl.ds(base + (c + 1) * CHUNK, CHUNK)], idx[nb])
            gathers[c + 1] = start_gather(nb)
        for d in gathers[c]:
            d.wait()
        writes[c] = pltpu.async_copy(
            rows[b], out_hbm.at[pl.ds(base + c * CHUNK, CHUNK)], wsem)
    writes[n_chunks - 2].wait()
    writes[n_chunks - 1].wait()


def kernel(x, table):
    batch, n_fields = x.shape
    n_flat = batch * n_fields
    x_flat = x.reshape(n_flat).astype(jnp.int32)

    mesh = plsc.VectorSubcoreMesh(core_axis_name="c", subcore_axis_name="s")
    emb = pl.kernel(
        _emb_body,
        out_type=jax.ShapeDtypeStruct((n_flat, EMB_DIM), jnp.float32),
        mesh=mesh,
        scratch_types=[
            pltpu.VMEM((CHUNK,), jnp.int32),
            pltpu.VMEM((CHUNK,), jnp.int32),
            pltpu.VMEM((CHUNK, EMB_DIM), jnp.float32),
            pltpu.VMEM((CHUNK, EMB_DIM), jnp.float32),
            pltpu.SemaphoreType.DMA,
            pltpu.SemaphoreType.DMA,
        ],
        compiler_params=pltpu.CompilerParams(use_tc_tiling_on_sc=False),
    )
    out_flat = emb(x_flat, table)
    return out_flat.reshape(batch, n_fields, EMB_DIM)
